# baseline (speedup 1.0000x reference)
import jax
import jax.numpy as jnp
from jax import lax
from jax.experimental import pallas as pl
from jax.experimental.pallas import tpu as pltpu
from jax.experimental.pallas import tpu_sc as plsc

_R = 7
_K = 2 * _R + 1
_H = 64
_W = 64
_HW = _H * _W
_NC = 2
_NS = 16
_L = 16
_SLOTS = 96
_SW = _SLOTS * _W
_JPT = _H // _NS
_IPC = _H // _NC
_CBT = _HW // 128 // _NS


def _sc_body(b_hbm, out_hbm, bvm, rows8, s0_sh, s64_sh, sem):
    c = lax.axis_index("c")
    s = lax.axis_index("s")
    pltpu.sync_copy(b_hbm, bvm.at[pl.ds(0, _K * _K)])

    ph = s // 8
    g = s % 8
    slot0 = 56 - ph - (1 - c) * 32
    zeros16 = jnp.zeros((_L,), jnp.float32)

    def zero_body(k, carry):
        for jl in range(8):
            for u in range(8):
                rows8[jl, pl.ds(k * 128 + u * _L, _L)] = zeros16
        return carry

    lax.fori_loop(0, _SW // 128, zero_body, 0)

    lane = lax.iota(jnp.int32, _L)

    def fill_body(t, carry):
        jl = t // _K
        a = t % _K
        j = g * 8 + jl
        row = bvm[pl.ds(a * _K, _L)]
        for m in range(_W // _L):
            q = m * _L + lane
            b = q - j + _R
            inb = jnp.logical_and(b >= 0, b < _K)
            bcl = jnp.clip(b, 0, _K - 1)
            vals = jnp.where(inb, row.at[bcl].get(mode="promise_in_bounds"), 0.0)
            rows8[jl, pl.ds((slot0 + a) * _W + m * _L, _L)] = vals
        return carry

    lax.fori_loop(0, 8 * _K, fill_body, 0)

    @pl.when(ph == 0)
    def _():
        pltpu.sync_copy(rows8, s0_sh.at[pl.ds(pl.multiple_of(g * 8, 8), 8), :])

    @pl.when(ph == 1)
    def _():
        pltpu.sync_copy(rows8, s64_sh.at[pl.ds(pl.multiple_of(g * 8, 8), 8), :])

    plsc.subcore_barrier()

    def fire_body(ii, carry):
        i_ev = c * _IPC + 2 * ii
        base = (15 - ii) * 128
        for k in range(_CBT):
            cb = s * _CBT + k
            srcc = pl.multiple_of(base + cb * 128, 128)
            dst_col = pl.multiple_of(cb * 128, 128)
            pltpu.async_copy(
                s64_sh.at[:, pl.ds(srcc, 128)],
                out_hbm.at[
                    pl.ds(pl.multiple_of(i_ev * _W, 64), _H),
                    pl.ds(dst_col, 128),
                ],
                sem,
            )
            pltpu.async_copy(
                s0_sh.at[:, pl.ds(srcc, 128)],
                out_hbm.at[
                    pl.ds(pl.multiple_of((i_ev + 1) * _W, 64), _H),
                    pl.ds(dst_col, 128),
                ],
                sem,
            )
        return carry

    lax.fori_loop(0, _IPC // 2, fire_body, 0)

    for _ in range(2):
        pltpu.make_async_copy(
            out_hbm.at[pl.ds(0, _H), pl.ds(0, 4096)],
            s0_sh.at[:, pl.ds(0, 4096)],
            sem,
        ).wait()


def kernel(feat_shape, biases):
    del feat_shape
    mesh = plsc.VectorSubcoreMesh(
        core_axis_name="c", subcore_axis_name="s",
        num_cores=_NC, num_subcores=_NS,
    )
    run = pl.kernel(
        _sc_body,
        out_type=jax.ShapeDtypeStruct((_HW, _HW), jnp.float32),
        mesh=mesh,
        scratch_types=[
            pltpu.VMEM((_K * _K + _L,), jnp.float32),
            pltpu.VMEM((8, _SW), jnp.float32),
            pltpu.VMEM_SHARED((_H, _SW), jnp.float32),
            pltpu.VMEM_SHARED((_H, _SW), jnp.float32),
            pltpu.SemaphoreType.DMA,
        ],
    )
    return run(biases.reshape(_K * _K))


# speedup vs baseline: 1.0049x; 1.0049x over previous
import jax
import jax.numpy as jnp
from jax import lax
from jax.experimental import pallas as pl
from jax.experimental.pallas import tpu as pltpu
from jax.experimental.pallas import tpu_sc as plsc

_R = 7
_K = 2 * _R + 1
_H = 64
_W = 64
_HW = _H * _W
_NC = 2
_NS = 16
_L = 16
_SLOTS = 96
_SW = _SLOTS * _W
_JPT = _H // _NS
_IPC = _H // _NC
_CBT = _HW // 128 // _NS


def _sc_body(b_hbm, out_hbm, bvm, rows8, band8, s0_sh, s64_sh, sem, sem2):
    c = lax.axis_index("c")
    s = lax.axis_index("s")
    pltpu.sync_copy(b_hbm, bvm.at[pl.ds(0, _K * _K)])

    ph = s // 8
    g = s % 8
    slot0 = 56 - ph - (1 - c) * 32
    zeros16 = jnp.zeros((_L,), jnp.float32)

    def zero_body(k, carry):
        for jl in range(8):
            for u in range(8):
                rows8[jl, pl.ds(k * 128 + u * _L, _L)] = zeros16
        return carry

    lax.fori_loop(0, _SW // 128, zero_body, 0)

    rg = pl.ds(pl.multiple_of(g * 8, 8), 8)

    @pl.when(ph == 0)
    def _():
        pltpu.async_copy(rows8, s0_sh.at[rg, :], sem2)

    @pl.when(ph == 1)
    def _():
        pltpu.async_copy(rows8, s64_sh.at[rg, :], sem2)

    def bzero_body(k, carry):
        for jl in range(8):
            for u in range(8):
                band8[jl, pl.ds(k * 128 + u * _L, _L)] = zeros16
        return carry

    lax.fori_loop(0, 16 * _W // 128, bzero_body, 0)

    lane = lax.iota(jnp.int32, _L)
    off0 = (slot0 & 1) * _W
    ws = pl.multiple_of((slot0 - (slot0 & 1)) * _W, 128)

    def fill_body(t, carry):
        jl = t // _K
        a = t % _K
        j = g * 8 + jl
        row = bvm[pl.ds(a * _K, _L)]
        for m in range(_W // _L):
            q = m * _L + lane
            b = q - j + _R
            inb = jnp.logical_and(b >= 0, b < _K)
            bcl = jnp.clip(b, 0, _K - 1)
            vals = jnp.where(inb, row.at[bcl].get(mode="promise_in_bounds"), 0.0)
            band8[jl, pl.ds(off0 + a * _W + m * _L, _L)] = vals
        return carry

    lax.fori_loop(0, 8 * _K, fill_body, 0)

    @pl.when(ph == 0)
    def _():
        pltpu.make_async_copy(rows8, s0_sh.at[rg, :], sem2).wait()
        pltpu.sync_copy(band8, s0_sh.at[rg, pl.ds(ws, 16 * _W)])

    @pl.when(ph == 1)
    def _():
        pltpu.make_async_copy(rows8, s64_sh.at[rg, :], sem2).wait()
        pltpu.sync_copy(band8, s64_sh.at[rg, pl.ds(ws, 16 * _W)])

    plsc.subcore_barrier()

    def fire_body(ii, carry):
        i_ev = c * _IPC + 2 * ii
        base = (15 - ii) * 128
        for k in range(_CBT):
            cb = s * _CBT + k
            srcc = pl.multiple_of(base + cb * 128, 128)
            dst_col = pl.multiple_of(cb * 128, 128)
            pltpu.async_copy(
                s64_sh.at[:, pl.ds(srcc, 128)],
                out_hbm.at[
                    pl.ds(pl.multiple_of(i_ev * _W, 64), _H),
                    pl.ds(dst_col, 128),
                ],
                sem,
            )
            pltpu.async_copy(
                s0_sh.at[:, pl.ds(srcc, 128)],
                out_hbm.at[
                    pl.ds(pl.multiple_of((i_ev + 1) * _W, 64), _H),
                    pl.ds(dst_col, 128),
                ],
                sem,
            )
        return carry

    lax.fori_loop(0, _IPC // 2, fire_body, 0)

    for _ in range(2):
        pltpu.make_async_copy(
            out_hbm.at[pl.ds(0, _H), pl.ds(0, 4096)],
            s0_sh.at[:, pl.ds(0, 4096)],
            sem,
        ).wait()


def kernel(feat_shape, biases):
    del feat_shape
    mesh = plsc.VectorSubcoreMesh(
        core_axis_name="c", subcore_axis_name="s",
        num_cores=_NC, num_subcores=_NS,
    )
    run = pl.kernel(
        _sc_body,
        out_type=jax.ShapeDtypeStruct((_HW, _HW), jnp.float32),
        mesh=mesh,
        scratch_types=[
            pltpu.VMEM((_K * _K + _L,), jnp.float32),
            pltpu.VMEM((8, _SW), jnp.float32),
            pltpu.VMEM((8, 16 * _W), jnp.float32),
            pltpu.VMEM_SHARED((_H, _SW), jnp.float32),
            pltpu.VMEM_SHARED((_H, _SW), jnp.float32),
            pltpu.SemaphoreType.DMA,
            pltpu.SemaphoreType.DMA,
        ],
    )
    return run(biases.reshape(_K * _K))
